# baseline (speedup 1.0000x reference)
import functools

import jax
import jax.numpy as jnp
from jax import lax
from jax.experimental import pallas as pl
from jax.experimental.pallas import tpu as pltpu
from jax.experimental.pallas import tpu_sc as plsc

N_NODE = 50000
E_EDGE = 400000
D = 128

NS = 16
EP_TILE = 25088
E_PAD = EP_TILE * NS
CE = 3136
N_CHUNK = EP_TILE // CE
G = 64
Z_ROWS = 784

RP_S = 12544
NP_S = 4
NPAD_S = RP_S * NP_S
RT_S = RP_S // NS
GS = 64
CAP_S = CE + GS

RP_C = 12544
NP_C = 4
NPAD_C = RP_C * NP_C
RT_C = RP_C // NS
NBUF_C = 2
CAP_C = CE + NBUF_C * G


def _sc_mesh():
    return plsc.VectorSubcoreMesh(core_axis_name="c", subcore_axis_name="s")


def _compact_chunk(dst_hbm, st_dst, cdst, row_lo, base_e, rpass, pad_mult,
                   extra=None):
    z16 = jnp.zeros((16,), jnp.int32)
    o16 = jnp.full((16,), 1, jnp.int32)
    pltpu.sync_copy(dst_hbm.at[pl.ds(base_e, CE)], st_dst)
    if extra is not None:
        src_hbm, st_src, csrc = extra
        pltpu.sync_copy(src_hbm.at[pl.ds(base_e, CE)], st_src)

    def body(i, n):
        dv = st_dst[pl.ds(i * 16, 16)]
        dl = dv - row_lo
        m = jnp.logical_and(dl >= z16, dl < z16 + rpass)
        mi = jnp.where(m, o16, z16)
        pos = n + plsc.cumsum(mi) - 1
        plsc.store_scatter(cdst, [pos], dl, mask=m)
        if extra is not None:
            sv = st_src[pl.ds(i * 16, 16)]
            plsc.store_scatter(extra[2], [pos], sv, mask=m)
        return n + jnp.sum(mi)

    n0 = lax.fori_loop(0, CE // 16, body, jnp.int32(0))
    dummy_d = jnp.full((16,), rpass, jnp.int32)
    dummy_s = jnp.zeros((16,), jnp.int32)
    lanes = lax.iota(jnp.int32, 16)
    for j in range(pad_mult // 16):
        plsc.store_scatter(cdst, [n0 + j * 16 + lanes], dummy_d)
        if extra is not None:
            plsc.store_scatter(extra[2], [n0 + j * 16 + lanes], dummy_s)
    return ((n0 + (pad_mult - 1)) // pad_mult) * pad_mult


def _segsum_sc(feat, src, dst, zeros128):
    @functools.partial(
        pl.kernel,
        mesh=_sc_mesh(),
        out_type=jax.ShapeDtypeStruct((NPAD_S, D), jnp.float32),
        scratch_types=[
            pltpu.VMEM_SHARED((RP_S + 8, D), jnp.float32),
            pltpu.VMEM((CE,), jnp.int32),
            pltpu.VMEM((CE,), jnp.int32),
            pltpu.VMEM((CAP_S,), jnp.int32),
            pltpu.VMEM((CAP_S,), jnp.int32),
            pltpu.VMEM((GS,), jnp.int32),
            pltpu.VMEM((GS,), jnp.int32),
            pltpu.VMEM((GS,), jnp.int32),
            pltpu.VMEM((GS, D), jnp.float32),
            pltpu.VMEM((GS, D), jnp.float32),
            pltpu.SemaphoreType.DMA,
            pltpu.SemaphoreType.DMA,
        ],
        compiler_params=pltpu.CompilerParams(needs_layout_passes=False),
    )
    def k(src_hbm, dst_hbm, feat_hbm, zero_hbm, out_hbm,
          acc, st_src, st_dst, csrc, cdst, gidx, didx0, didx1,
          gbuf0, gbuf1, gsem, ssem):
        didx = (didx0, didx1)
        gbuf = (gbuf0, gbuf1)
        cid = lax.axis_index("c")
        sid = lax.axis_index("s")
        for p in range(NP_S // 2):
            pass_id = cid * (NP_S // 2) + p
            row_lo = pass_id * RP_S
            pltpu.sync_copy(zero_hbm.at[pl.ds(0, RT_S)],
                            acc.at[pl.ds(sid * RT_S, RT_S)])
            plsc.subcore_barrier()

            def chunk_body(c, st):
                base_e = sid * EP_TILE + c * CE
                n = _compact_chunk(dst_hbm, st_dst, cdst, row_lo, base_e,
                                   RP_S, 2 * GS,
                                   extra=(src_hbm, st_src, csrc))

                def gs_body(k2, st):
                    for b in range(2):
                        kk = k2 * 2 + b
                        @pl.when(st == 1)
                        def _():
                            pltpu.make_async_copy(
                                gbuf[b], acc.at[didx[b]], ssem).wait()
                        for j in range(GS // 16):
                            gidx[pl.ds(j * 16, 16)] = (
                                csrc[pl.ds(kk * GS + j * 16, 16)])
                            didx[b][pl.ds(j * 16, 16)] = (
                                cdst[pl.ds(kk * GS + j * 16, 16)])
                        pltpu.async_copy(feat_hbm.at[gidx], gbuf[b],
                                         gsem).wait()
                        pltpu.async_copy(gbuf[b], acc.at[didx[b]], ssem,
                                         add=True)
                    return jnp.int32(1)

                return lax.fori_loop(0, n // (2 * GS), gs_body, st)

            st = lax.fori_loop(0, N_CHUNK, chunk_body, jnp.int32(0))
            @pl.when(st == 1)
            def _():
                for b in range(2):
                    pltpu.make_async_copy(gbuf[b], acc.at[didx[b]],
                                          ssem).wait()
            plsc.subcore_barrier()
            out_row = row_lo + sid * RT_S
            pltpu.sync_copy(acc.at[pl.ds(sid * RT_S, RT_S)],
                            out_hbm.at[pl.ds(out_row, RT_S)])
            plsc.subcore_barrier()

    return k(src, dst, feat, zeros128)


def _counts_sc(dst, zeros128, ones128):
    @functools.partial(
        pl.kernel,
        mesh=_sc_mesh(),
        out_type=jax.ShapeDtypeStruct((NPAD_C, D), jnp.float32),
        scratch_types=(
            [pltpu.VMEM_SHARED((RP_C + 8, D), jnp.float32),
             pltpu.VMEM((CE,), jnp.int32),
             pltpu.VMEM((CAP_C,), jnp.int32),
             pltpu.VMEM((G, D), jnp.float32)]
            + [pltpu.VMEM((G,), jnp.int32)] * NBUF_C
            + [pltpu.SemaphoreType.DMA] * NBUF_C
        ),
        compiler_params=pltpu.CompilerParams(needs_layout_passes=False),
    )
    def k(dst_hbm, zero_hbm, one_hbm, out_hbm, *scr):
        acc, st_dst, cdst, ones = scr[:4]
        didx = scr[4:4 + NBUF_C]
        ssem = scr[4 + NBUF_C:4 + 2 * NBUF_C]
        cid = lax.axis_index("c")
        sid = lax.axis_index("s")
        pltpu.sync_copy(one_hbm, ones)
        for p in range(NP_C // 2):
            pass_id = cid * (NP_C // 2) + p
            row_lo = pass_id * RP_C
            pltpu.sync_copy(zero_hbm.at[pl.ds(0, RT_C)],
                            acc.at[pl.ds(sid * RT_C, RT_C)])
            plsc.subcore_barrier()

            def chunk_body(c, st):
                base_e = sid * EP_TILE + c * CE
                n = _compact_chunk(dst_hbm, st_dst, cdst, row_lo, base_e,
                                   RP_C, NBUF_C * G)

                def grp(k2, st):
                    for b in range(NBUF_C):
                        kk = k2 * NBUF_C + b
                        @pl.when(st == 1)
                        def _():
                            pltpu.make_async_copy(
                                ones, acc.at[didx[b]], ssem[b]).wait()
                        for j in range(G // 16):
                            didx[b][pl.ds(j * 16, 16)] = (
                                cdst[pl.ds(kk * G + j * 16, 16)])
                        pltpu.async_copy(ones, acc.at[didx[b]], ssem[b],
                                         add=True)
                    return jnp.int32(1)

                return lax.fori_loop(0, n // (NBUF_C * G), grp, st)

            st = lax.fori_loop(0, N_CHUNK, chunk_body, jnp.int32(0))
            @pl.when(st == 1)
            def _():
                for b in range(NBUF_C):
                    pltpu.make_async_copy(ones, acc.at[didx[b]],
                                          ssem[b]).wait()
            plsc.subcore_barrier()
            out_row = row_lo + sid * RT_C
            pltpu.sync_copy(acc.at[pl.ds(sid * RT_C, RT_C)],
                            out_hbm.at[pl.ds(out_row, RT_C)])
            plsc.subcore_barrier()

    return k(dst, zeros128, ones128)


B_ROWS = 1000
_GRID = N_NODE // B_ROWS


def _blk(shape):
    return pl.BlockSpec(shape, lambda i: (i, 0))


def _rep(shape):
    return pl.BlockSpec(shape, lambda i: (0, 0))


def _mm(a, b):
    return jax.lax.dot_general(a, b, (((1,), (0,)), ((), ())),
                               preferred_element_type=jnp.float32)


def _inproj_tc(x, W, b):
    def k(x_ref, w_ref, b_ref, o_ref):
        o_ref[...] = _mm(x_ref[...], w_ref[...]) + b_ref[...]

    return pl.pallas_call(
        k,
        grid=(_GRID,),
        in_specs=[_blk((B_ROWS, D)), _rep((D, D)), _rep((1, D))],
        out_specs=_blk((B_ROWS, D)),
        out_shape=jax.ShapeDtypeStruct((N_NODE, D), jnp.float32),
    )(x, W, b.reshape(1, D))


def _combine_tc(S, cnt, h, W_l, W_r, b):
    def k(s_ref, c_ref, h_ref, wl_ref, wr_ref, b_ref, o_ref):
        inv = 1.0 / jnp.maximum(c_ref[...], 1.0)
        acc = _mm(s_ref[...] * inv, wl_ref[...])
        acc = acc + _mm(h_ref[...], wr_ref[...])
        o_ref[...] = jnp.maximum(acc + b_ref[...], 0.0)

    return pl.pallas_call(
        k,
        grid=(_GRID,),
        in_specs=[_blk((B_ROWS, D)), _blk((B_ROWS, D)), _blk((B_ROWS, D)),
                  _rep((D, D)), _rep((D, D)), _rep((1, D))],
        out_specs=_blk((B_ROWS, D)),
        out_shape=jax.ShapeDtypeStruct((N_NODE, D), jnp.float32),
    )(S, cnt, h, W_l, W_r, b.reshape(1, D))


def _combine_out_tc(S, cnt, h, W_l, W_r, b, W_o, b_o):
    def k(s_ref, c_ref, h_ref, wl_ref, wr_ref, b_ref, wo_ref, bo_ref, o_ref):
        inv = 1.0 / jnp.maximum(c_ref[...], 1.0)
        acc = _mm(s_ref[...] * inv, wl_ref[...])
        acc = acc + _mm(h_ref[...], wr_ref[...])
        t = jnp.maximum(acc + b_ref[...], 0.0)
        y = _mm(t, wo_ref[...]) + bo_ref[...]
        nrm = jnp.sqrt(jnp.sum(y * y, axis=1, keepdims=True))
        o_ref[...] = y / jnp.maximum(nrm, 1e-12)

    return pl.pallas_call(
        k,
        grid=(_GRID,),
        in_specs=[_blk((B_ROWS, D)), _blk((B_ROWS, D)), _blk((B_ROWS, D)),
                  _rep((D, D)), _rep((D, D)), _rep((1, D)),
                  _rep((D, D)), _rep((1, D))],
        out_specs=_blk((B_ROWS, D)),
        out_shape=jax.ShapeDtypeStruct((N_NODE, D), jnp.float32),
    )(S, cnt, h, W_l, W_r, b.reshape(1, D), W_o, b_o.reshape(1, D))


def kernel(x_project, x_company, edge_index_p2c, edge_index_c2p,
           W_in_project, b_in_project, W_in_company, b_in_company,
           W_l0_p2c, b_l0_p2c, W_r0_p2c, W_l0_c2p, b_l0_c2p, W_r0_c2p,
           W_l1_p2c, b_l1_p2c, W_r1_p2c, W_l1_c2p, b_l1_c2p, W_r1_c2p,
           W_out_project, b_out_project, W_out_company, b_out_company):
    pad = E_PAD - E_EDGE
    src_p2c = jnp.pad(edge_index_p2c[0].astype(jnp.int32), (0, pad))
    dst_p2c = jnp.pad(edge_index_p2c[1].astype(jnp.int32), (0, pad),
                      constant_values=-1)
    src_c2p = jnp.pad(edge_index_c2p[0].astype(jnp.int32), (0, pad))
    dst_c2p = jnp.pad(edge_index_c2p[1].astype(jnp.int32), (0, pad),
                      constant_values=-1)
    zeros128 = jnp.zeros((Z_ROWS, D), jnp.float32)
    ones128 = jnp.ones((G, D), jnp.float32)

    cnt_c = _counts_sc(dst_p2c, zeros128, ones128)
    cnt_p = _counts_sc(dst_c2p, zeros128, ones128)

    h_p = _inproj_tc(x_project, W_in_project, b_in_project)
    h_c = _inproj_tc(x_company, W_in_company, b_in_company)

    S_c = _segsum_sc(h_p, src_p2c, dst_p2c, zeros128)
    S_p = _segsum_sc(h_c, src_c2p, dst_c2p, zeros128)
    h_c1 = _combine_tc(S_c, cnt_c, h_c, W_l0_p2c, W_r0_p2c, b_l0_p2c)
    h_p1 = _combine_tc(S_p, cnt_p, h_p, W_l0_c2p, W_r0_c2p, b_l0_c2p)

    S_c = _segsum_sc(h_p1, src_p2c, dst_p2c, zeros128)
    S_p = _segsum_sc(h_c1, src_c2p, dst_c2p, zeros128)
    z_c = _combine_out_tc(S_c, cnt_c, h_c1, W_l1_p2c, W_r1_p2c, b_l1_p2c,
                          W_out_company, b_out_company)
    z_p = _combine_out_tc(S_p, cnt_p, h_p1, W_l1_c2p, W_r1_c2p, b_l1_c2p,
                          W_out_project, b_out_project)
    return (z_p, z_c)


# speedup vs baseline: 1.9295x; 1.9295x over previous
import functools

import jax
import jax.numpy as jnp
from jax import lax
from jax.experimental import pallas as pl
from jax.experimental.pallas import tpu as pltpu
from jax.experimental.pallas import tpu_sc as plsc

N_NODE = 50000
E_EDGE = 400000
D = 128

NS = 16
EP_TILE = 25088
E_PAD = EP_TILE * NS
CE = 3136
N_CHUNK = EP_TILE // CE
G = 64
Z_ROWS = 784

RP_S = 12544
NP_S = 4
NPAD_S = RP_S * NP_S
RT_S = RP_S // NS
GS = 64
CAP_S = CE + GS

RP_C = 12544
NP_C = 4
NPAD_C = RP_C * NP_C
RT_C = RP_C // NS
NBUF_C = 2
CAP_C = CE + NBUF_C * G


def _sc_mesh():
    return plsc.VectorSubcoreMesh(core_axis_name="c", subcore_axis_name="s")


def _compact_chunk(dst_hbm, st_dst, cdst, row_lo, base_e, rpass, pad_mult,
                   extra=None):
    z16 = jnp.zeros((16,), jnp.int32)
    o16 = jnp.full((16,), 1, jnp.int32)
    pltpu.sync_copy(dst_hbm.at[pl.ds(base_e, CE)], st_dst)
    if extra is not None:
        src_hbm, st_src, csrc = extra
        pltpu.sync_copy(src_hbm.at[pl.ds(base_e, CE)], st_src)

    def body(i, n):
        dv = st_dst[pl.ds(i * 16, 16)]
        dl = dv - row_lo
        m = jnp.logical_and(dl >= z16, dl < z16 + rpass)
        mi = jnp.where(m, o16, z16)
        pos = n + plsc.cumsum(mi) - 1
        plsc.store_scatter(cdst, [pos], dl, mask=m)
        if extra is not None:
            sv = st_src[pl.ds(i * 16, 16)]
            plsc.store_scatter(extra[2], [pos], sv, mask=m)
        return n + jnp.sum(mi)

    n0 = lax.fori_loop(0, CE // 16, body, jnp.int32(0))
    dummy_d = jnp.full((16,), rpass, jnp.int32)
    dummy_s = jnp.zeros((16,), jnp.int32)
    lanes = lax.iota(jnp.int32, 16)
    for j in range(pad_mult // 16):
        plsc.store_scatter(cdst, [n0 + j * 16 + lanes], dummy_d)
        if extra is not None:
            plsc.store_scatter(extra[2], [n0 + j * 16 + lanes], dummy_s)
    return ((n0 + (pad_mult - 1)) // pad_mult) * pad_mult


def _segsum_sc(feat, src, dst, zeros128):
    @functools.partial(
        pl.kernel,
        mesh=_sc_mesh(),
        out_type=jax.ShapeDtypeStruct((NPAD_S, D), jnp.float32),
        scratch_types=[
            pltpu.VMEM_SHARED((RP_S + 8, D), jnp.float32),
            pltpu.VMEM((CE,), jnp.int32),
            pltpu.VMEM((CE,), jnp.int32),
            pltpu.VMEM((CAP_S,), jnp.int32),
            pltpu.VMEM((CAP_S,), jnp.int32),
            pltpu.VMEM((GS,), jnp.int32),
            pltpu.VMEM((GS,), jnp.int32),
            pltpu.VMEM((GS, D), jnp.float32),
            pltpu.SemaphoreType.DMA,
        ],
        compiler_params=pltpu.CompilerParams(needs_layout_passes=False),
    )
    def k(src_hbm, dst_hbm, feat_hbm, zero_hbm, out_hbm,
          acc, st_src, st_dst, csrc, cdst, gidx, didx, gbuf, gsem):
        cid = lax.axis_index("c")
        sid = lax.axis_index("s")
        for p in range(NP_S // 2):
            pass_id = cid * (NP_S // 2) + p
            row_lo = pass_id * RP_S
            pltpu.sync_copy(zero_hbm.at[pl.ds(0, RT_S)],
                            acc.at[pl.ds(sid * RT_S, RT_S)])
            plsc.subcore_barrier()

            def chunk_body(c, _):
                base_e = sid * EP_TILE + c * CE
                n = _compact_chunk(dst_hbm, st_dst, cdst, row_lo, base_e,
                                   RP_S, GS,
                                   extra=(src_hbm, st_src, csrc))

                def gs_body(ci, _):
                    for j in range(GS // 16):
                        gidx[pl.ds(j * 16, 16)] = (
                            csrc[pl.ds(ci * GS + j * 16, 16)])
                        didx[pl.ds(j * 16, 16)] = (
                            cdst[pl.ds(ci * GS + j * 16, 16)])
                    pltpu.async_copy(feat_hbm.at[gidx], gbuf, gsem).wait()
                    pltpu.sync_copy(gbuf, acc.at[didx], add=True)
                    return 0

                lax.fori_loop(0, n // GS, gs_body, 0)
                return 0

            lax.fori_loop(0, N_CHUNK, chunk_body, 0)
            plsc.subcore_barrier()
            out_row = row_lo + sid * RT_S
            pltpu.sync_copy(acc.at[pl.ds(sid * RT_S, RT_S)],
                            out_hbm.at[pl.ds(out_row, RT_S)])
            plsc.subcore_barrier()

    return k(src, dst, feat, zeros128)


def _counts_sc(dst, zeros128, ones128):
    @functools.partial(
        pl.kernel,
        mesh=_sc_mesh(),
        out_type=jax.ShapeDtypeStruct((NPAD_C, D), jnp.float32),
        scratch_types=(
            [pltpu.VMEM_SHARED((RP_C + 8, D), jnp.float32),
             pltpu.VMEM((CE,), jnp.int32),
             pltpu.VMEM((CAP_C,), jnp.int32),
             pltpu.VMEM((G, D), jnp.float32)]
            + [pltpu.VMEM((G,), jnp.int32)] * NBUF_C
            + [pltpu.SemaphoreType.DMA] * NBUF_C
        ),
        compiler_params=pltpu.CompilerParams(needs_layout_passes=False),
    )
    def k(dst_hbm, zero_hbm, one_hbm, out_hbm, *scr):
        acc, st_dst, cdst, ones = scr[:4]
        didx = scr[4:4 + NBUF_C]
        ssem = scr[4 + NBUF_C:4 + 2 * NBUF_C]
        cid = lax.axis_index("c")
        sid = lax.axis_index("s")
        pltpu.sync_copy(one_hbm, ones)
        for p in range(NP_C // 2):
            pass_id = cid * (NP_C // 2) + p
            row_lo = pass_id * RP_C
            pltpu.sync_copy(zero_hbm.at[pl.ds(0, RT_C)],
                            acc.at[pl.ds(sid * RT_C, RT_C)])
            plsc.subcore_barrier()

            def chunk_body(c, st):
                base_e = sid * EP_TILE + c * CE
                n = _compact_chunk(dst_hbm, st_dst, cdst, row_lo, base_e,
                                   RP_C, NBUF_C * G)

                def grp(k2, st):
                    for b in range(NBUF_C):
                        kk = k2 * NBUF_C + b
                        @pl.when(st == 1)
                        def _():
                            pltpu.make_async_copy(
                                ones, acc.at[didx[b]], ssem[b]).wait()
                        for j in range(G // 16):
                            didx[b][pl.ds(j * 16, 16)] = (
                                cdst[pl.ds(kk * G + j * 16, 16)])
                        pltpu.async_copy(ones, acc.at[didx[b]], ssem[b],
                                         add=True)
                    return jnp.int32(1)

                return lax.fori_loop(0, n // (NBUF_C * G), grp, st)

            st = lax.fori_loop(0, N_CHUNK, chunk_body, jnp.int32(0))
            @pl.when(st == 1)
            def _():
                for b in range(NBUF_C):
                    pltpu.make_async_copy(ones, acc.at[didx[b]],
                                          ssem[b]).wait()
            plsc.subcore_barrier()
            out_row = row_lo + sid * RT_C
            pltpu.sync_copy(acc.at[pl.ds(sid * RT_C, RT_C)],
                            out_hbm.at[pl.ds(out_row, RT_C)])
            plsc.subcore_barrier()

    return k(dst, zeros128, ones128)


B_ROWS = 1000
_GRID = N_NODE // B_ROWS


def _blk(shape):
    return pl.BlockSpec(shape, lambda i: (i, 0))


def _rep(shape):
    return pl.BlockSpec(shape, lambda i: (0, 0))


def _mm(a, b):
    return jax.lax.dot_general(a, b, (((1,), (0,)), ((), ())),
                               preferred_element_type=jnp.float32)


def _inproj_tc(x, W, b):
    def k(x_ref, w_ref, b_ref, o_ref):
        o_ref[...] = _mm(x_ref[...], w_ref[...]) + b_ref[...]

    return pl.pallas_call(
        k,
        grid=(_GRID,),
        in_specs=[_blk((B_ROWS, D)), _rep((D, D)), _rep((1, D))],
        out_specs=_blk((B_ROWS, D)),
        out_shape=jax.ShapeDtypeStruct((N_NODE, D), jnp.float32),
    )(x, W, b.reshape(1, D))


def _combine_tc(S, cnt, h, W_l, W_r, b):
    def k(s_ref, c_ref, h_ref, wl_ref, wr_ref, b_ref, o_ref):
        inv = 1.0 / jnp.maximum(c_ref[...], 1.0)
        acc = _mm(s_ref[...] * inv, wl_ref[...])
        acc = acc + _mm(h_ref[...], wr_ref[...])
        o_ref[...] = jnp.maximum(acc + b_ref[...], 0.0)

    return pl.pallas_call(
        k,
        grid=(_GRID,),
        in_specs=[_blk((B_ROWS, D)), _blk((B_ROWS, D)), _blk((B_ROWS, D)),
                  _rep((D, D)), _rep((D, D)), _rep((1, D))],
        out_specs=_blk((B_ROWS, D)),
        out_shape=jax.ShapeDtypeStruct((N_NODE, D), jnp.float32),
    )(S, cnt, h, W_l, W_r, b.reshape(1, D))


def _combine_out_tc(S, cnt, h, W_l, W_r, b, W_o, b_o):
    def k(s_ref, c_ref, h_ref, wl_ref, wr_ref, b_ref, wo_ref, bo_ref, o_ref):
        inv = 1.0 / jnp.maximum(c_ref[...], 1.0)
        acc = _mm(s_ref[...] * inv, wl_ref[...])
        acc = acc + _mm(h_ref[...], wr_ref[...])
        t = jnp.maximum(acc + b_ref[...], 0.0)
        y = _mm(t, wo_ref[...]) + bo_ref[...]
        nrm = jnp.sqrt(jnp.sum(y * y, axis=1, keepdims=True))
        o_ref[...] = y / jnp.maximum(nrm, 1e-12)

    return pl.pallas_call(
        k,
        grid=(_GRID,),
        in_specs=[_blk((B_ROWS, D)), _blk((B_ROWS, D)), _blk((B_ROWS, D)),
                  _rep((D, D)), _rep((D, D)), _rep((1, D)),
                  _rep((D, D)), _rep((1, D))],
        out_specs=_blk((B_ROWS, D)),
        out_shape=jax.ShapeDtypeStruct((N_NODE, D), jnp.float32),
    )(S, cnt, h, W_l, W_r, b.reshape(1, D), W_o, b_o.reshape(1, D))


def kernel(x_project, x_company, edge_index_p2c, edge_index_c2p,
           W_in_project, b_in_project, W_in_company, b_in_company,
           W_l0_p2c, b_l0_p2c, W_r0_p2c, W_l0_c2p, b_l0_c2p, W_r0_c2p,
           W_l1_p2c, b_l1_p2c, W_r1_p2c, W_l1_c2p, b_l1_c2p, W_r1_c2p,
           W_out_project, b_out_project, W_out_company, b_out_company):
    pad = E_PAD - E_EDGE
    src_p2c = jnp.pad(edge_index_p2c[0].astype(jnp.int32), (0, pad))
    dst_p2c = jnp.pad(edge_index_p2c[1].astype(jnp.int32), (0, pad),
                      constant_values=-1)
    src_c2p = jnp.pad(edge_index_c2p[0].astype(jnp.int32), (0, pad))
    dst_c2p = jnp.pad(edge_index_c2p[1].astype(jnp.int32), (0, pad),
                      constant_values=-1)
    zeros128 = jnp.zeros((Z_ROWS, D), jnp.float32)
    ones128 = jnp.ones((G, D), jnp.float32)

    cnt_c = _counts_sc(dst_p2c, zeros128, ones128)
    cnt_p = _counts_sc(dst_c2p, zeros128, ones128)

    h_p = _inproj_tc(x_project, W_in_project, b_in_project)
    h_c = _inproj_tc(x_company, W_in_company, b_in_company)

    S_c = _segsum_sc(h_p, src_p2c, dst_p2c, zeros128)
    S_p = _segsum_sc(h_c, src_c2p, dst_c2p, zeros128)
    h_c1 = _combine_tc(S_c, cnt_c, h_c, W_l0_p2c, W_r0_p2c, b_l0_p2c)
    h_p1 = _combine_tc(S_p, cnt_p, h_p, W_l0_c2p, W_r0_c2p, b_l0_c2p)

    S_c = _segsum_sc(h_p1, src_p2c, dst_p2c, zeros128)
    S_p = _segsum_sc(h_c1, src_c2p, dst_c2p, zeros128)
    z_c = _combine_out_tc(S_c, cnt_c, h_c1, W_l1_p2c, W_r1_p2c, b_l1_p2c,
                          W_out_company, b_out_company)
    z_p = _combine_out_tc(S_p, cnt_p, h_p1, W_l1_c2p, W_r1_c2p, b_l1_c2p,
                          W_out_project, b_out_project)
    return (z_p, z_c)
